# baseline (speedup 1.0000x reference)
import functools

import jax
import jax.numpy as jnp
from jax.experimental import pallas as pl
from jax.experimental.pallas import tpu as pltpu

_B = 16
_C = 64
_HW = 64 * 64
_K = 1024
_T = 4096
_NJ = _HW // _T


def _vq_block(z_ref, emb_ref, gmat_ref, quant_ref, idx_ref, loss_ref):
    zb = z_ref[0]
    emb = emb_ref[...]
    scores = jax.lax.dot_general(
        emb, zb, (((0,), (0,)), ((), ())),
        preferred_element_type=jnp.float32)
    h = 0.5 * jnp.sum(emb * emb, axis=0)
    score = scores - h[:, None]
    m = jnp.max(score, axis=0)
    onehot = (score >= m[None, :]).astype(jnp.float32)
    qa = jax.lax.dot_general(
        gmat_ref[...], onehot, (((1,), (0,)), ((), ())),
        preferred_element_type=jnp.float32)
    quant = qa[:_C]
    quant_ref[0] = quant
    idx_ref[0, 0] = (qa[_C] + 0.5).astype(jnp.int32)
    diff = quant - zb
    loss_ref[...] = jnp.sum(diff * diff).reshape(1, 1, 1)


@jax.jit
def kernel(z, embedding):
    commitment_cost = 0.25
    z3 = z.reshape(_B, _C, _HW)
    iota8 = jax.lax.broadcasted_iota(jnp.float32, (8, _K), 1)
    gmat = jnp.concatenate([embedding, iota8], axis=0)
    grid = (_B, _NJ)
    quant, idx, loss_parts = pl.pallas_call(
        _vq_block,
        grid=grid,
        in_specs=[
            pl.BlockSpec((1, _C, _T), lambda b, j: (b, 0, j)),
            pl.BlockSpec((_C, _K), lambda b, j: (0, 0)),
            pl.BlockSpec((_C + 8, _K), lambda b, j: (0, 0)),
        ],
        out_specs=[
            pl.BlockSpec((1, _C, _T), lambda b, j: (b, 0, j)),
            pl.BlockSpec((1, 1, _T), lambda b, j: (b * _NJ + j, 0, 0)),
            pl.BlockSpec((1, 1, 1), lambda b, j: (b * _NJ + j, 0, 0)),
        ],
        out_shape=[
            jax.ShapeDtypeStruct((_B, _C, _HW), jnp.float32),
            jax.ShapeDtypeStruct((_B * _NJ, 1, _T), jnp.int32),
            jax.ShapeDtypeStruct((_B * _NJ, 1, 1), jnp.float32),
        ],
        compiler_params=pltpu.CompilerParams(
            dimension_semantics=("parallel", "parallel")),
    )(z3, embedding, gmat)
    quantized_out = quant.reshape(z.shape)
    encoding_indices = idx.reshape(_B, 64, 64)
    loss = (1.0 + commitment_cost) * jnp.sum(loss_parts) / z.size
    return (quantized_out, loss, encoding_indices)


# speedup vs baseline: 1.0232x; 1.0232x over previous
import functools

import jax
import jax.numpy as jnp
from jax.experimental import pallas as pl
from jax.experimental.pallas import tpu as pltpu

_B = 16
_C = 64
_HW = 64 * 64
_K = 1024
_T = 4096
_NJ = _HW // _T


def _vq_block(z_ref, emb_ref, quant_ref, idx_ref, loss_ref):
    zb = z_ref[0]
    emb = emb_ref[...]
    scores = jax.lax.dot_general(
        emb, zb, (((0,), (0,)), ((), ())),
        preferred_element_type=jnp.float32)
    h = 0.5 * jnp.sum(emb * emb, axis=0)
    score = scores - h[:, None]
    idx = jnp.argmax(score, axis=0)
    onehot = (jax.lax.broadcasted_iota(jnp.int32, (_K, _T), 0)
              == idx[None, :]).astype(jnp.float32)
    quant = jax.lax.dot_general(
        emb, onehot, (((1,), (0,)), ((), ())),
        preferred_element_type=jnp.float32)
    quant_ref[0] = quant
    idx_ref[0, 0] = idx
    diff = quant - zb
    loss_ref[...] = jnp.sum(diff * diff).reshape(1, 1, 1)


@jax.jit
def kernel(z, embedding):
    commitment_cost = 0.25
    z3 = z.reshape(_B, _C, _HW)
    grid = (_B, _NJ)
    quant, idx, loss_parts = pl.pallas_call(
        _vq_block,
        grid=grid,
        in_specs=[
            pl.BlockSpec((1, _C, _T), lambda b, j: (b, 0, j)),
            pl.BlockSpec((_C, _K), lambda b, j: (0, 0)),
        ],
        out_specs=[
            pl.BlockSpec((1, _C, _T), lambda b, j: (b, 0, j)),
            pl.BlockSpec((1, 1, _T), lambda b, j: (b * _NJ + j, 0, 0)),
            pl.BlockSpec((1, 1, 1), lambda b, j: (b * _NJ + j, 0, 0)),
        ],
        out_shape=[
            jax.ShapeDtypeStruct((_B, _C, _HW), jnp.float32),
            jax.ShapeDtypeStruct((_B * _NJ, 1, _T), jnp.int32),
            jax.ShapeDtypeStruct((_B * _NJ, 1, 1), jnp.float32),
        ],
        compiler_params=pltpu.CompilerParams(
            dimension_semantics=("arbitrary", "arbitrary")),
    )(z3, embedding)
    quantized_out = quant.reshape(z.shape)
    encoding_indices = idx.reshape(_B, 64, 64)
    loss = (1.0 + commitment_cost) * jnp.sum(loss_parts) / z.size
    return (quantized_out, loss, encoding_indices)


# speedup vs baseline: 1.0488x; 1.0250x over previous
import functools

import jax
import jax.numpy as jnp
from jax.experimental import pallas as pl
from jax.experimental.pallas import tpu as pltpu

_B = 16
_C = 64
_HW = 64 * 64
_K = 1024


def _vq_pipeline(z_hbm, emb_ref, quant_hbm, idx_hbm, loss_ref,
                 zbuf, qbuf, ibuf, in_sem, outq_sem, outi_sem):
    emb = emb_ref[...]
    h = 0.5 * jnp.sum(emb * emb, axis=0)

    def in_copy(i):
        return pltpu.make_async_copy(
            z_hbm.at[i], zbuf.at[i % 2], in_sem.at[i % 2])

    def outq_copy(i):
        return pltpu.make_async_copy(
            qbuf.at[i % 2], quant_hbm.at[i], outq_sem.at[i % 2])

    def outi_copy(i):
        return pltpu.make_async_copy(
            ibuf.at[i % 2], idx_hbm.at[i], outi_sem.at[i % 2])

    in_copy(0).start()
    loss_acc = jnp.zeros((_HW,), jnp.float32)
    for i in range(_B):
        if i + 1 < _B:
            in_copy(i + 1).start()
        in_copy(i).wait()
        zb = zbuf[i % 2]
        scores = jax.lax.dot_general(
            emb, zb, (((0,), (0,)), ((), ())),
            preferred_element_type=jnp.float32)
        score = scores - h[:, None]
        idx = jnp.argmax(score, axis=0)
        onehot = (jax.lax.broadcasted_iota(jnp.int32, (_K, _HW), 0)
                  == idx[None, :]).astype(jnp.float32)
        quant = jax.lax.dot_general(
            emb, onehot, (((1,), (0,)), ((), ())),
            preferred_element_type=jnp.float32)
        if i >= 2:
            outq_copy(i - 2).wait()
            outi_copy(i - 2).wait()
        qbuf[i % 2] = quant
        ibuf[i % 2, 0] = idx
        outq_copy(i).start()
        outi_copy(i).start()
        diff = quant - zb
        loss_acc = loss_acc + jnp.sum(diff * diff, axis=0)
    outq_copy(_B - 2).wait()
    outi_copy(_B - 2).wait()
    outq_copy(_B - 1).wait()
    outi_copy(_B - 1).wait()
    loss_ref[0] = loss_acc


@jax.jit
def kernel(z, embedding):
    commitment_cost = 0.25
    z3 = z.reshape(_B, _C, _HW)
    quant, idx, loss_parts = pl.pallas_call(
        _vq_pipeline,
        in_specs=[
            pl.BlockSpec(memory_space=pl.ANY),
            pl.BlockSpec(memory_space=pltpu.VMEM),
        ],
        out_specs=[
            pl.BlockSpec(memory_space=pl.ANY),
            pl.BlockSpec(memory_space=pl.ANY),
            pl.BlockSpec(memory_space=pltpu.VMEM),
        ],
        out_shape=[
            jax.ShapeDtypeStruct((_B, _C, _HW), jnp.float32),
            jax.ShapeDtypeStruct((_B, 1, _HW), jnp.int32),
            jax.ShapeDtypeStruct((1, _HW), jnp.float32),
        ],
        scratch_shapes=[
            pltpu.VMEM((2, _C, _HW), jnp.float32),
            pltpu.VMEM((2, _C, _HW), jnp.float32),
            pltpu.VMEM((2, 1, _HW), jnp.int32),
            pltpu.SemaphoreType.DMA((2,)),
            pltpu.SemaphoreType.DMA((2,)),
            pltpu.SemaphoreType.DMA((2,)),
        ],
    )(z3, embedding)
    quantized_out = quant.reshape(z.shape)
    encoding_indices = idx.reshape(_B, 64, 64)
    loss = (1.0 + commitment_cost) * jnp.sum(loss_parts) / z.size
    return (quantized_out, loss, encoding_indices)
